# Initial kernel scaffold; baseline (speedup 1.0000x reference)
import jax
import jax.numpy as jnp
from jax.experimental import pallas as pl


def kernel(x, Wg, We, be):
    raise NotImplementedError("write your pallas kernel here")


# speedup vs baseline: 2.3652x; 2.3652x over previous
import functools

import jax
import jax.numpy as jnp
from jax.experimental import pallas as pl
from jax.experimental.pallas import tpu as pltpu

_INTERPRET = False

K = 2
BLK = 256


def _gate_body(x_ref, wg_ref, w1_ref, w2_ref, a1_ref, a2_ref):
    logits = jnp.dot(x_ref[...], wg_ref[...], preferred_element_type=jnp.float32)
    ncol = logits.shape[1]
    col = jax.lax.broadcasted_iota(jnp.int32, logits.shape, 1)
    m1 = jnp.max(logits, axis=1)
    a1 = jnp.min(jnp.where(logits == m1[:, None], col, ncol), axis=1)
    masked = jnp.where(col == a1[:, None], -jnp.inf, logits)
    m2 = jnp.max(masked, axis=1)
    a2 = jnp.min(jnp.where(masked == m2[:, None], col, ncol), axis=1)
    e2 = jnp.exp(m2 - m1)
    denom = 1.0 + e2
    w1_ref[...] = 1.0 / denom
    w2_ref[...] = e2 / denom
    a1_ref[...] = a1
    a2_ref[...] = a2


def _gate(x, Wg):
    tok = x.shape[0]
    return pl.pallas_call(
        _gate_body,
        out_shape=(
            jax.ShapeDtypeStruct((tok,), jnp.float32),
            jax.ShapeDtypeStruct((tok,), jnp.float32),
            jax.ShapeDtypeStruct((tok,), jnp.int32),
            jax.ShapeDtypeStruct((tok,), jnp.int32),
        ),
        interpret=_INTERPRET,
    )(x, Wg)


def _gmm_body(e_ref, b_ref, lo_ref, hi_ref, first_ref,
              xs_ref, we_ref, be_ref, ws_ref, out_ref):
    t = pl.program_id(0)
    rows = jax.lax.broadcasted_iota(jnp.int32, (BLK, 1), 0) + b_ref[t] * BLK
    mask = (rows >= lo_ref[t]) & (rows < hi_ref[t])
    wm = jnp.where(mask, ws_ref[0, 0, :][:, None], 0.0)
    acc = jax.lax.dot_general(
        xs_ref[...], we_ref[0], (((1,), (1,)), ((), ())),
        preferred_element_type=jnp.float32)
    contrib = wm * (acc + be_ref[0, 0, :][None, :])

    @pl.when(first_ref[t] == 1)
    def _():
        out_ref[...] = contrib

    @pl.when(first_ref[t] == 0)
    def _():
        out_ref[...] += contrib


def _gmm(xs, We, be, ws, expert_t, block_t, lo_t, hi_t, first_t, num_items):
    a, d = xs.shape
    e = We.shape[0]
    tiles_m = a // BLK
    be3 = be.reshape(e, 1, d)
    ws3 = ws.reshape(tiles_m, 1, BLK)
    grid_spec = pltpu.PrefetchScalarGridSpec(
        num_scalar_prefetch=5,
        grid=(num_items,),
        in_specs=[
            pl.BlockSpec((BLK, d), lambda t, e_, b_, *_: (b_[t], 0)),
            pl.BlockSpec((1, d, d), lambda t, e_, b_, *_: (e_[t], 0, 0)),
            pl.BlockSpec((1, 1, d), lambda t, e_, b_, *_: (e_[t], 0, 0)),
            pl.BlockSpec((1, 1, BLK), lambda t, e_, b_, *_: (b_[t], 0, 0)),
        ],
        out_specs=pl.BlockSpec((BLK, d), lambda t, e_, b_, *_: (b_[t], 0)),
    )
    return pl.pallas_call(
        _gmm_body,
        grid_spec=grid_spec,
        out_shape=jax.ShapeDtypeStruct((a, d), jnp.float32),
        interpret=_INTERPRET,
    )(expert_t, block_t, lo_t, hi_t, first_t, xs, We, be3, ws3)


def _routing_metadata(e_sorted, e, a, num_items):
    i32 = jnp.int32
    off = jnp.searchsorted(e_sorted, jnp.arange(e + 1, dtype=i32)).astype(i32)
    sizes = off[1:] - off[:-1]
    first_blk = off[:e] // BLK
    last_blk = jnp.where(sizes > 0, off[1:] - 1, off[:e]) // BLK
    nspan = jnp.where(sizes > 0, last_blk - first_blk + 1, 0).astype(i32)
    cum = jnp.cumsum(nspan).astype(i32)
    total = cum[e - 1]
    t = jnp.arange(num_items, dtype=i32)
    g_t = jnp.minimum(jnp.searchsorted(cum, t, side="right").astype(i32), e - 1)
    blk_t = first_blk[g_t] + t - (cum[g_t] - nspan[g_t])
    g_last = jnp.minimum(
        jnp.searchsorted(cum, total - 1, side="right").astype(i32), e - 1)
    valid = t < total
    expert_t = jnp.where(valid, g_t, g_last)
    block_t = jnp.where(valid, blk_t, last_blk[g_last])
    lo_t = jnp.where(valid, jnp.maximum(off[expert_t], block_t * BLK), 0)
    hi_t = jnp.where(valid, jnp.minimum(off[expert_t + 1], (block_t + 1) * BLK), 0)
    first_t = jnp.concatenate(
        [jnp.ones((1,), i32), (block_t[1:] != block_t[:-1]).astype(i32)])
    return expert_t, block_t, lo_t, hi_t, first_t


@jax.jit
def kernel(x, Wg, We, be):
    tok, d = x.shape
    e = We.shape[0]
    a = tok * K
    num_items = a // BLK + e - 1

    w1, w2, a1, a2 = _gate(x, Wg)
    weights = jnp.stack([w1, w2], axis=1)
    selected = jnp.stack([a1, a2], axis=1)

    e_flat = selected.reshape(-1)
    order = jnp.argsort(e_flat, stable=True).astype(jnp.int32)
    e_sorted = e_flat[order]
    meta = _routing_metadata(e_sorted, e, a, num_items)

    tok_sorted = order // K
    ws = weights.reshape(-1)[order]
    xs = x[tok_sorted]

    ys = _gmm(xs, We, be, ws, *meta, num_items)

    inv = jnp.argsort(order).astype(jnp.int32)
    out = ys[inv[0::2]] + ys[inv[1::2]]
    return out
